# Initial kernel scaffold; baseline (speedup 1.0000x reference)
import jax
import jax.numpy as jnp
import numpy as np
from jax import lax
from jax.experimental import pallas as pl
from jax.experimental.pallas import tpu as pltpu
from jax.experimental.pallas import tpu_sc as plsc

VOCAB = 100000
D = 128
B = 1024
L = 200

_NUM_CORES = 2
_NUM_SUBCORES = 16
_NW = _NUM_CORES * _NUM_SUBCORES
_ROWS = B * L
_ROWS_PER_W = _ROWS // _NW
_CHUNK = 128
_NCHUNK = _ROWS_PER_W // _CHUNK


def _sin_pe(max_len, d_model):
    pos = np.arange(max_len, dtype=np.float32)[:, None]
    div = np.exp(
        np.arange(0, d_model, 2, dtype=np.float32) * -(np.log(10000.0) / d_model)
    )
    pe = np.zeros((max_len, d_model), dtype=np.float32)
    pe[:, 0::2] = np.sin(pos * div)
    pe[:, 1::2] = np.cos(pos * div)
    return pe


_PE = _sin_pe(L, D)


def _embed_kernel(tok_idx_hbm, combo_idx_hbm, tok_table_hbm, combo_hbm, out_hbm,
                  idx_t, idx_c, rows_t, rows_c, sem_t, sem_c):
    wid = lax.axis_index("s") * _NUM_CORES + lax.axis_index("c")
    base = wid * _ROWS_PER_W

    def chunk_body(i, _):
        off = base + i * _CHUNK
        pltpu.sync_copy(tok_idx_hbm.at[pl.ds(off, _CHUNK)], idx_t)
        pltpu.sync_copy(combo_idx_hbm.at[pl.ds(off, _CHUNK)], idx_c)
        cp_t = pltpu.async_copy(tok_table_hbm.at[idx_t], rows_t, sem_t)
        cp_c = pltpu.async_copy(combo_hbm.at[idx_c], rows_c, sem_c)
        cp_t.wait()
        cp_c.wait()

        def add_body(j, _):
            s = pl.ds(j * 16, 16)
            flat_t = rows_t.reshape(_CHUNK * D)
            flat_c = rows_c.reshape(_CHUNK * D)
            flat_t[s] = flat_t[s] + flat_c[s]
            return ()

        lax.fori_loop(0, _CHUNK * D // 16, add_body, (), unroll=8)
        pltpu.sync_copy(rows_t, out_hbm.at[pl.ds(off, _CHUNK)])
        return ()

    lax.fori_loop(0, _NCHUNK, chunk_body, ())


@jax.jit
def kernel(sequence, segment_label, token_table, seg_table):
    tok_idx = sequence.reshape(-1).astype(jnp.int32)
    pos = jnp.arange(L, dtype=jnp.int32)
    combo_idx = (segment_label.astype(jnp.int32) * L + pos[None, :]).reshape(-1)
    combo = (seg_table[:, None, :] + jnp.asarray(_PE)[None, :, :]).reshape(3 * L, D)

    mesh = plsc.VectorSubcoreMesh(core_axis_name="c", subcore_axis_name="s")
    run = pl.kernel(
        _embed_kernel,
        mesh=mesh,
        out_type=jax.ShapeDtypeStruct((_ROWS, D), jnp.float32),
        scratch_types=[
            pltpu.VMEM((_CHUNK,), jnp.int32),
            pltpu.VMEM((_CHUNK,), jnp.int32),
            pltpu.VMEM((_CHUNK, D), jnp.float32),
            pltpu.VMEM((_CHUNK, D), jnp.float32),
            pltpu.SemaphoreType.DMA,
            pltpu.SemaphoreType.DMA,
        ],
    )
    out = run(tok_idx, combo_idx, token_table, combo)
    return out.reshape(B, L, D)


# speedup vs baseline: 5.3262x; 5.3262x over previous
import jax
import jax.numpy as jnp
import numpy as np
from jax import lax
from jax.experimental import pallas as pl
from jax.experimental.pallas import tpu as pltpu
from jax.experimental.pallas import tpu_sc as plsc

VOCAB = 100000
D = 128
B = 1024
L = 200

_NUM_CORES = 2
_NUM_SUBCORES = 16
_NW = _NUM_CORES * _NUM_SUBCORES
_ROWS = B * L
_ROWS_PER_W = _ROWS // _NW
_CHUNK = 128
_NCHUNK = _ROWS_PER_W // _CHUNK


def _sin_pe(max_len, d_model):
    pos = np.arange(max_len, dtype=np.float32)[:, None]
    div = np.exp(
        np.arange(0, d_model, 2, dtype=np.float32) * -(np.log(10000.0) / d_model)
    )
    pe = np.zeros((max_len, d_model), dtype=np.float32)
    pe[:, 0::2] = np.sin(pos * div)
    pe[:, 1::2] = np.cos(pos * div)
    return pe


_PE = _sin_pe(L, D)


def _embed_kernel(tok_idx_hbm, combo_idx_hbm, tok_table_hbm, combo_hbm, out_hbm,
                  idx_t, idx_c, rows_t, rows_c, sem_t, sem_c):
    wid = lax.axis_index("s") * _NUM_CORES + lax.axis_index("c")
    base = wid * _ROWS_PER_W

    def chunk_body(i, _):
        off = base + i * _CHUNK
        pltpu.sync_copy(tok_idx_hbm.at[pl.ds(off, _CHUNK)], idx_t)
        pltpu.sync_copy(combo_idx_hbm.at[pl.ds(off, _CHUNK)], idx_c)
        cp_t = pltpu.async_copy(tok_table_hbm.at[idx_t], rows_t, sem_t)
        cp_c = pltpu.async_copy(combo_hbm.at[idx_c], rows_c, sem_c)
        cp_t.wait()
        cp_c.wait()

        def add_body(j, _):
            for k in range(D // 16):
                s = pl.ds(k * 16, 16)
                rows_t[j, s] = rows_t[j, s] + rows_c[j, s]
            return ()

        lax.fori_loop(0, _CHUNK, add_body, ())
        pltpu.sync_copy(rows_t, out_hbm.at[pl.ds(off, _CHUNK)])
        return ()

    lax.fori_loop(0, _NCHUNK, chunk_body, ())


@jax.jit
def kernel(sequence, segment_label, token_table, seg_table):
    tok_idx = sequence.reshape(-1).astype(jnp.int32)
    pos = jnp.arange(L, dtype=jnp.int32)
    combo_idx = (segment_label.astype(jnp.int32) * L + pos[None, :]).reshape(-1)
    combo = (seg_table[:, None, :] + jnp.asarray(_PE)[None, :, :]).reshape(3 * L, D)

    mesh = plsc.VectorSubcoreMesh(core_axis_name="c", subcore_axis_name="s")
    run = pl.kernel(
        _embed_kernel,
        mesh=mesh,
        out_type=jax.ShapeDtypeStruct((_ROWS, D), jnp.float32),
        scratch_types=[
            pltpu.VMEM((_CHUNK,), jnp.int32),
            pltpu.VMEM((_CHUNK,), jnp.int32),
            pltpu.VMEM((_CHUNK, D), jnp.float32),
            pltpu.VMEM((_CHUNK, D), jnp.float32),
            pltpu.SemaphoreType.DMA,
            pltpu.SemaphoreType.DMA,
        ],
    )
    out = run(tok_idx, combo_idx, token_table, combo)
    return out.reshape(B, L, D)
